# baseline (speedup 1.0000x reference)
import jax
import jax.numpy as jnp
from jax.experimental import pallas as pl

_C = 32
_V = _C * (_C + 1) // 2


def _body(x_ref, o_ref):
    for r in range(_C):
        s = r * (r + 1) // 2
        o_ref[:, s:s + r + 1] = x_ref[:, _C * r:_C * r + r + 1]


def kernel(inputs):
    T, N, B, C, C2 = inputs.shape
    M = T * N * B
    x = inputs.reshape(M, C * C2)
    R = 512
    out = pl.pallas_call(
        _body,
        grid=(M // R,),
        in_specs=[pl.BlockSpec((R, C * C2), lambda i: (i, 0))],
        out_specs=pl.BlockSpec((R, _V), lambda i: (i, 0)),
        out_shape=jax.ShapeDtypeStruct((M, _V), jnp.float32),
    )(x)
    return out.reshape(T, N, B, _V)
